# baseline (speedup 1.0000x reference)
import functools

import jax
import jax.numpy as jnp
from jax import lax
from jax.experimental import pallas as pl
from jax.experimental.pallas import tpu as pltpu
from jax.experimental.pallas import tpu_sc as plsc

EMBED = 128
BATCH = 4096
SEQ = 200

NUM_CORES = 2
NUM_SUBCORES = 16
NUM_WORKERS = NUM_CORES * NUM_SUBCORES
BPW = BATCH // NUM_WORKERS
CHUNK = BPW * SEQ
LANES = 16

N_SLOTS = 5
SUBROUNDS = SEQ // N_SLOTS
BODIES = SUBROUNDS // 2
MASK_PER_SR = CHUNK // SUBROUNDS
MASK_VECS_PER_SR = MASK_PER_SR // LANES

_mesh = plsc.VectorSubcoreMesh(core_axis_name="c", subcore_axis_name="s")


@functools.partial(
    pl.kernel,
    mesh=_mesh,
    out_type=(
        jax.ShapeDtypeStruct((SEQ * BATCH, EMBED), jnp.float32),
        jax.ShapeDtypeStruct((BATCH * SEQ,), jnp.float32),
    ),
    scratch_types=[
        pltpu.VMEM((CHUNK,), jnp.int32),
        pltpu.VMEM((MASK_PER_SR,), jnp.float32),
        [pltpu.VMEM((BPW,), jnp.int32)] * (2 * N_SLOTS),
        [pltpu.VMEM((BPW, EMBED), jnp.float32)] * N_SLOTS,
        [pltpu.SemaphoreType.DMA] * N_SLOTS,
        [pltpu.SemaphoreType.DMA] * N_SLOTS,
        pltpu.SemaphoreType.DMA,
    ],
    compiler_params=pltpu.CompilerParams(needs_layout_passes=False),
)
def _lookup(x_hbm, table_hbm, out_hbm, mask_hbm,
            x_v, mstage, idx, rows, gsem, wsem, msem):
    wid = lax.axis_index("s") * NUM_CORES + lax.axis_index("c")
    base = wid * CHUNK
    b0 = wid * BPW

    pltpu.sync_copy(x_hbm.at[pl.ds(base, CHUNK)], x_v)

    def build_idx(s, idx_ref):
        for g in range(BPW // LANES):
            bvec = lax.iota(jnp.int32, LANES) + (g * LANES)
            vals = plsc.load_gather(x_v, [bvec * SEQ + s])
            idx_ref[pl.ds(g * LANES, LANES)] = vals

    def out_slice(s):
        return out_hbm.at[pl.ds(s * BATCH + b0, BPW), :]

    def mask_slice(sr):
        return mask_hbm.at[pl.ds(base + sr * MASK_PER_SR, MASK_PER_SR)]

    def mask_chunk(sr):
        m0 = sr * MASK_PER_SR
        for t in range(MASK_VECS_PER_SR):
            v = x_v[pl.ds(m0 + t * LANES, LANES)]
            mstage[pl.ds(t * LANES, LANES)] = jnp.where(
                v != 0, jnp.float32(1.0), jnp.float32(0.0))
        pltpu.async_copy(mstage, mask_slice(sr), msem)

    def drain_mask():
        pltpu.make_async_copy(mstage, mask_slice(0), msem).wait()

    for j in range(N_SLOTS):
        build_idx(j, idx[j])

    def body(i, c):
        sA = (2 * i) * N_SLOTS
        sB = sA + N_SLOTS

        for j in range(N_SLOTS):
            @pl.when(i > 0)
            def _():
                pltpu.make_async_copy(rows[j], out_slice(sA + j), wsem[j]).wait()
            pltpu.async_copy(table_hbm.at[idx[j]], rows[j], gsem[j])

        for j in range(N_SLOTS):
            build_idx(sB + j, idx[N_SLOTS + j])

        @pl.when(i > 0)
        def _():
            drain_mask()
        mask_chunk(2 * i)

        for j in range(N_SLOTS):
            pltpu.make_async_copy(table_hbm.at[idx[j]], rows[j], gsem[j]).wait()
            pltpu.async_copy(rows[j], out_slice(sA + j), wsem[j])

        for j in range(N_SLOTS):
            pltpu.make_async_copy(rows[j], out_slice(sB + j), wsem[j]).wait()
            pltpu.async_copy(table_hbm.at[idx[N_SLOTS + j]], rows[j], gsem[j])

        @pl.when(i < BODIES - 1)
        def _():
            for j in range(N_SLOTS):
                build_idx(sB + N_SLOTS + j, idx[j])

        drain_mask()
        mask_chunk(2 * i + 1)

        for j in range(N_SLOTS):
            pltpu.make_async_copy(
                table_hbm.at[idx[N_SLOTS + j]], rows[j], gsem[j]).wait()
            pltpu.async_copy(rows[j], out_slice(sB + j), wsem[j])
        return c

    lax.fori_loop(0, BODIES, body, 0)

    for j in range(N_SLOTS):
        pltpu.make_async_copy(rows[j], out_slice(SEQ - N_SLOTS + j),
                              wsem[j]).wait()
    drain_mask()


def kernel(x, table):
    out_rows, mask_flat = _lookup(x.reshape(-1), table)
    return (out_rows.reshape(SEQ, BATCH, EMBED), mask_flat.reshape(BATCH, SEQ))


# speedup vs baseline: 1.0242x; 1.0242x over previous
import functools

import jax
import jax.numpy as jnp
from jax import lax
from jax.experimental import pallas as pl
from jax.experimental.pallas import tpu as pltpu
from jax.experimental.pallas import tpu_sc as plsc

EMBED = 128
BATCH = 4096
SEQ = 200

NUM_CORES = 2
NUM_SUBCORES = 16
NUM_WORKERS = NUM_CORES * NUM_SUBCORES
BPW = BATCH // NUM_WORKERS
HPW = BPW // 2
CHUNK = BPW * SEQ
LANES = 16

N_SLOTS = 5
ROUNDS = SEQ // N_SLOTS
MASK_PER_ROUND = CHUNK // ROUNDS
MASK_VECS_PER_ROUND = MASK_PER_ROUND // LANES

_mesh = plsc.VectorSubcoreMesh(core_axis_name="c", subcore_axis_name="s")


@functools.partial(
    pl.kernel,
    mesh=_mesh,
    out_type=(
        jax.ShapeDtypeStruct((SEQ * BATCH, EMBED), jnp.float32),
        jax.ShapeDtypeStruct((BATCH * SEQ,), jnp.float32),
    ),
    scratch_types=[
        pltpu.VMEM((CHUNK,), jnp.int32),
        pltpu.VMEM((MASK_PER_ROUND,), jnp.float32),
        [pltpu.VMEM((BPW,), jnp.int32)] * N_SLOTS,
        [pltpu.VMEM((BPW, EMBED), jnp.float32)] * N_SLOTS,
        [pltpu.SemaphoreType.DMA] * N_SLOTS,
        [pltpu.SemaphoreType.DMA] * N_SLOTS,
        [pltpu.SemaphoreType.DMA] * N_SLOTS,
        pltpu.SemaphoreType.DMA,
    ],
    compiler_params=pltpu.CompilerParams(needs_layout_passes=False),
)
def _lookup(x_hbm, table_hbm, out_hbm, mask_hbm,
            x_v, mstage, idx, rows, gsemA, gsemB, wsem, msem):
    wid = lax.axis_index("s") * NUM_CORES + lax.axis_index("c")
    base = wid * CHUNK
    b0 = wid * BPW

    pltpu.sync_copy(x_hbm.at[pl.ds(base, CHUNK)], x_v)

    def build_idx(s, idx_ref):
        for g in range(BPW // LANES):
            bvec = lax.iota(jnp.int32, LANES) + (g * LANES)
            vals = plsc.load_gather(x_v, [bvec * SEQ + s])
            idx_ref[pl.ds(g * LANES, LANES)] = vals

    def gather_halves(j):
        a = pltpu.make_async_copy(
            table_hbm.at[idx[j].at[pl.ds(0, HPW)]],
            rows[j].at[pl.ds(0, HPW), :], gsemA[j])
        b = pltpu.make_async_copy(
            table_hbm.at[idx[j].at[pl.ds(HPW, HPW)]],
            rows[j].at[pl.ds(HPW, HPW), :], gsemB[j])
        return a, b

    def out_slice(s):
        return out_hbm.at[pl.ds(s * BATCH + b0, BPW), :]

    def mask_slice(i):
        return mask_hbm.at[pl.ds(base + i * MASK_PER_ROUND, MASK_PER_ROUND)]

    def body(i, c):
        for j in range(N_SLOTS):
            s = i * N_SLOTS + j

            @pl.when(i > 0)
            def _():
                pltpu.make_async_copy(rows[j], out_slice(s), wsem[j]).wait()

            build_idx(s, idx[j])
            ca, cb = gather_halves(j)
            ca.start()
            cb.start()

        @pl.when(i > 0)
        def _():
            pltpu.make_async_copy(mstage, mask_slice(i), msem).wait()

        m0 = i * MASK_PER_ROUND
        for t in range(MASK_VECS_PER_ROUND):
            v = x_v[pl.ds(m0 + t * LANES, LANES)]
            mstage[pl.ds(t * LANES, LANES)] = jnp.where(
                v != 0, jnp.float32(1.0), jnp.float32(0.0))
        pltpu.async_copy(mstage, mask_slice(i), msem)

        for j in range(N_SLOTS):
            s = i * N_SLOTS + j
            ca, cb = gather_halves(j)
            ca.wait()
            cb.wait()
            pltpu.async_copy(rows[j], out_slice(s), wsem[j])
        return c

    lax.fori_loop(0, ROUNDS, body, 0)

    for j in range(N_SLOTS):
        s = SEQ - N_SLOTS + j
        pltpu.make_async_copy(rows[j], out_slice(s), wsem[j]).wait()
    pltpu.make_async_copy(mstage, mask_slice(0), msem).wait()


def kernel(x, table):
    out_rows, mask_flat = _lookup(x.reshape(-1), table)
    return (out_rows.reshape(SEQ, BATCH, EMBED), mask_flat.reshape(BATCH, SEQ))
